# Initial kernel scaffold; baseline (speedup 1.0000x reference)
import jax
import jax.numpy as jnp
from jax.experimental import pallas as pl


def kernel(pcs, centroids):
    raise NotImplementedError("write your pallas kernel here")


# speedup vs baseline: 4.8222x; 4.8222x over previous
import functools

import jax
import jax.numpy as jnp
from jax import lax
from jax.experimental import pallas as pl
from jax.experimental.pallas import tpu as pltpu
from jax.experimental.pallas import tpu_sc as plsc

RADIUS2 = 0.2 * 0.2
MAXS = 32
LANES = 16
NWORKERS = 32


def _bf16_round(x):
    u = plsc.bitcast(x, jnp.uint32)
    rnd = (u >> 16) & jnp.uint32(1)
    u = (u + jnp.uint32(0x7FFF) + rnd) & jnp.uint32(0xFFFF0000)
    return plsc.bitcast(u, jnp.float32)


def _ball_body(px_h, py_h, pz_h, cx_h, cy_h, cz_h, out_h,
               px, py, pz, cx, cy, cz, outv, p2s):
    n = px.shape[0]
    m_per_w = cx.shape[0]
    n_chunks = n // LANES

    wid = lax.axis_index("s") * 2 + lax.axis_index("c")
    b = wid // 4

    pltpu.sync_copy(px_h.at[b], px)
    pltpu.sync_copy(py_h.at[b], py)
    pltpu.sync_copy(pz_h.at[b], pz)
    pltpu.sync_copy(cx_h.at[wid], cx)
    pltpu.sync_copy(cy_h.at[wid], cy)
    pltpu.sync_copy(cz_h.at[wid], cz)

    iota = jnp.arange(LANES, dtype=jnp.int32)
    zero = jnp.zeros((LANES,), jnp.int32)

    def prep_chunk(j, _):
        sl = pl.ds(j * LANES, LANES)
        xv, yv, zv = px[sl], py[sl], pz[sl]
        p2s[sl] = (xv * xv + yv * yv) + zv * zv
        px[sl] = _bf16_round(xv)
        py[sl] = _bf16_round(yv)
        pz[sl] = _bf16_round(zv)
        return 0

    lax.fori_loop(0, n_chunks, prep_chunk, 0)

    def per_centroid(i, _):
        rowbase = i * MAXS
        spl_i = lax.broadcast(i, (LANES,))
        cxi = plsc.load_gather(cx, [spl_i])
        cyi = plsc.load_gather(cy, [spl_i])
        czi = plsc.load_gather(cz, [spl_i])
        c2i = (cxi * cxi + cyi * cyi) + czi * czi
        cbx = _bf16_round(cxi)
        cby = _bf16_round(cyi)
        cbz = _bf16_round(czi)

        def per_chunk(j, cntv):
            off = j * LANES
            sl = pl.ds(off, LANES)
            cp = (cbx * px[sl] + cby * py[sl]) + cbz * pz[sl]
            d2 = (c2i + p2s[sl]) - 2.0 * cp
            m1 = d2 < RADIUS2
            m1i = m1.astype(jnp.int32)
            pos = cntv + plsc.cumsum(m1i) - m1i
            wm = m1 & (pos < MAXS)
            plsc.store_scatter(outv, [rowbase + pos], iota + off, mask=wm)
            return cntv + plsc.all_reduce_population_count(m1)

        cntv = lax.fori_loop(0, n_chunks, per_chunk, zero)

        firstv = plsc.load_gather(outv, [lax.broadcast(rowbase, (LANES,))])
        fillv = jnp.where(cntv > 0, firstv, 0)
        for s0 in range(0, MAXS, LANES):
            slots = iota + s0
            plsc.store_scatter(outv, [rowbase + slots], fillv,
                               mask=slots >= cntv)
        return 0

    lax.fori_loop(0, m_per_w, per_centroid, 0)
    pltpu.sync_copy(outv, out_h.at[wid])


def _ball_query_sc(px, py, pz, cx, cy, cz):
    n = px.shape[1]
    m_per_w = cx.shape[1]
    mesh = plsc.VectorSubcoreMesh(core_axis_name="c", subcore_axis_name="s")
    run = pl.kernel(
        _ball_body,
        out_type=jax.ShapeDtypeStruct((NWORKERS, m_per_w * MAXS), jnp.int32),
        mesh=mesh,
        compiler_params=pltpu.CompilerParams(needs_layout_passes=False),
        scratch_types=[
            pltpu.VMEM((n,), jnp.float32),
            pltpu.VMEM((n,), jnp.float32),
            pltpu.VMEM((n,), jnp.float32),
            pltpu.VMEM((m_per_w,), jnp.float32),
            pltpu.VMEM((m_per_w,), jnp.float32),
            pltpu.VMEM((m_per_w,), jnp.float32),
            pltpu.VMEM((m_per_w * MAXS,), jnp.int32),
            pltpu.VMEM((n,), jnp.float32),
        ],
    )
    return run(px, py, pz, cx, cy, cz)


def kernel(pcs, centroids):
    B, N, _ = pcs.shape
    M = centroids.shape[1]
    m_per_w = (B * M) // NWORKERS
    px = pcs[:, :, 0]
    py = pcs[:, :, 1]
    pz = pcs[:, :, 2]
    cx = centroids[:, :, 0].reshape(NWORKERS, m_per_w)
    cy = centroids[:, :, 1].reshape(NWORKERS, m_per_w)
    cz = centroids[:, :, 2].reshape(NWORKERS, m_per_w)
    out = _ball_query_sc(px, py, pz, cx, cy, cz)
    return out.reshape(B, M, MAXS).astype(jnp.int64)


# speedup vs baseline: 11.6045x; 2.4065x over previous
import functools

import jax
import jax.numpy as jnp
from jax import lax
from jax.experimental import pallas as pl
from jax.experimental.pallas import tpu as pltpu
from jax.experimental.pallas import tpu_sc as plsc

RADIUS2 = 0.2 * 0.2
MAXS = 32
LANES = 16
NWORKERS = 32
UNROLL = 4


def _bf16_round(x):
    u = plsc.bitcast(x, jnp.uint32)
    rnd = (u >> 16) & jnp.uint32(1)
    u = (u + jnp.uint32(0x7FFF) + rnd) & jnp.uint32(0xFFFF0000)
    return plsc.bitcast(u, jnp.float32)


def _ball_body(px_h, py_h, pz_h, cx_h, cy_h, cz_h, out_h,
               px, py, pz, cx, cy, cz, outv, p2s):
    n = px.shape[0]
    m_per_w = cx.shape[0]
    n_chunks = n // LANES

    wid = lax.axis_index("s") * 2 + lax.axis_index("c")
    b = wid // 4

    pltpu.sync_copy(px_h.at[b], px)
    pltpu.sync_copy(py_h.at[b], py)
    pltpu.sync_copy(pz_h.at[b], pz)
    pltpu.sync_copy(cx_h.at[wid], cx)
    pltpu.sync_copy(cy_h.at[wid], cy)
    pltpu.sync_copy(cz_h.at[wid], cz)

    iota = jnp.arange(LANES, dtype=jnp.int32)
    zero = jnp.zeros((LANES,), jnp.int32)

    def prep_chunk(j, _):
        sl = pl.ds(j * LANES, LANES)
        xv, yv, zv = px[sl], py[sl], pz[sl]
        p2s[sl] = (xv * xv + yv * yv) + zv * zv
        px[sl] = _bf16_round(xv)
        py[sl] = _bf16_round(yv)
        pz[sl] = _bf16_round(zv)
        return 0

    lax.fori_loop(0, n_chunks, prep_chunk, 0)

    def per_centroid(i, _):
        rowbase = i * MAXS
        spl_i = lax.broadcast(i, (LANES,))
        cxi = plsc.load_gather(cx, [spl_i])
        cyi = plsc.load_gather(cy, [spl_i])
        czi = plsc.load_gather(cz, [spl_i])
        c2i = (cxi * cxi + cyi * cyi) + czi * czi
        cbx = _bf16_round(cxi)
        cby = _bf16_round(cyi)
        cbz = _bf16_round(czi)

        def one_chunk(off, cntv):
            sl = pl.ds(off, LANES)
            cp = (cbx * px[sl] + cby * py[sl]) + cbz * pz[sl]
            d2 = (c2i + p2s[sl]) - 2.0 * cp
            m1 = d2 < RADIUS2
            m1i = m1.astype(jnp.int32)
            pos = cntv + plsc.cumsum(m1i) - m1i
            wm = m1 & (pos < MAXS)
            plsc.store_scatter(outv, [rowbase + pos], iota + off, mask=wm)
            return cntv + plsc.all_reduce_population_count(m1)

        def scan_cond(state):
            j, cntv = state
            return (j < n_chunks) & (jnp.max(cntv) < MAXS)

        def scan_body(state):
            j, cntv = state
            off = j * LANES
            for u in range(UNROLL):
                cntv = one_chunk(off + u * LANES, cntv)
            return j + UNROLL, cntv

        _, cntv = lax.while_loop(scan_cond, scan_body, (jnp.int32(0), zero))

        firstv = plsc.load_gather(outv, [lax.broadcast(rowbase, (LANES,))])
        fillv = jnp.where(cntv > 0, firstv, 0)
        for s0 in range(0, MAXS, LANES):
            slots = iota + s0
            plsc.store_scatter(outv, [rowbase + slots], fillv,
                               mask=slots >= cntv)
        return 0

    lax.fori_loop(0, m_per_w, per_centroid, 0)
    pltpu.sync_copy(outv, out_h.at[wid])


def _ball_query_sc(px, py, pz, cx, cy, cz):
    n = px.shape[1]
    m_per_w = cx.shape[1]
    mesh = plsc.VectorSubcoreMesh(core_axis_name="c", subcore_axis_name="s")
    run = pl.kernel(
        _ball_body,
        out_type=jax.ShapeDtypeStruct((NWORKERS, m_per_w * MAXS), jnp.int32),
        mesh=mesh,
        compiler_params=pltpu.CompilerParams(needs_layout_passes=False),
        scratch_types=[
            pltpu.VMEM((n,), jnp.float32),
            pltpu.VMEM((n,), jnp.float32),
            pltpu.VMEM((n,), jnp.float32),
            pltpu.VMEM((m_per_w,), jnp.float32),
            pltpu.VMEM((m_per_w,), jnp.float32),
            pltpu.VMEM((m_per_w,), jnp.float32),
            pltpu.VMEM((m_per_w * MAXS,), jnp.int32),
            pltpu.VMEM((n,), jnp.float32),
        ],
    )
    return run(px, py, pz, cx, cy, cz)


def kernel(pcs, centroids):
    B, N, _ = pcs.shape
    M = centroids.shape[1]
    m_per_w = (B * M) // NWORKERS
    px = pcs[:, :, 0]
    py = pcs[:, :, 1]
    pz = pcs[:, :, 2]
    cx = centroids[:, :, 0].reshape(NWORKERS, m_per_w)
    cy = centroids[:, :, 1].reshape(NWORKERS, m_per_w)
    cz = centroids[:, :, 2].reshape(NWORKERS, m_per_w)
    out = _ball_query_sc(px, py, pz, cx, cy, cz)
    return out.reshape(B, M, MAXS).astype(jnp.int64)


# speedup vs baseline: 32.2662x; 2.7805x over previous
import functools

import jax
import jax.numpy as jnp
from jax import lax
from jax.experimental import pallas as pl
from jax.experimental.pallas import tpu as pltpu
from jax.experimental.pallas import tpu_sc as plsc

RADIUS2 = 0.2 * 0.2
MAXS = 32
LANES = 16
NWORKERS = 32
UNROLL = 4


def _bf16_round(x):
    u = plsc.bitcast(x, jnp.uint32)
    rnd = (u >> 16) & jnp.uint32(1)
    u = (u + jnp.uint32(0x7FFF) + rnd) & jnp.uint32(0xFFFF0000)
    return plsc.bitcast(u, jnp.float32)


def _ball_body(px_h, py_h, pz_h, cx_h, cy_h, cz_h, out_h,
               px, py, pz, cx, cy, cz, outv, p2s):
    n = px.shape[0]
    m_per_w = cx.shape[0]
    n_chunks = n // LANES

    wid = lax.axis_index("s") * 2 + lax.axis_index("c")
    b = wid // 4

    pltpu.sync_copy(px_h.at[b], px)
    pltpu.sync_copy(py_h.at[b], py)
    pltpu.sync_copy(pz_h.at[b], pz)
    pltpu.sync_copy(cx_h.at[wid], cx)
    pltpu.sync_copy(cy_h.at[wid], cy)
    pltpu.sync_copy(cz_h.at[wid], cz)

    iota = jnp.arange(LANES, dtype=jnp.int32)
    zero = jnp.zeros((LANES,), jnp.int32)
    ones = jnp.ones((LANES,), jnp.int32)

    def prep_chunk(j, _):
        sl = pl.ds(j * LANES, LANES)
        xv, yv, zv = px[sl], py[sl], pz[sl]
        p2s[sl] = (xv * xv + yv * yv) + zv * zv
        px[sl] = _bf16_round(xv)
        py[sl] = _bf16_round(yv)
        pz[sl] = _bf16_round(zv)
        return 0

    lax.fori_loop(0, n_chunks, prep_chunk, 0)

    def per_centroid(i, _):
        rowbase = i * MAXS
        spl_i = lax.broadcast(i, (LANES,))
        cxi = plsc.load_gather(cx, [spl_i])
        cyi = plsc.load_gather(cy, [spl_i])
        czi = plsc.load_gather(cz, [spl_i])
        c2i = (cxi * cxi + cyi * cyi) + czi * czi
        cbx = _bf16_round(cxi)
        cby = _bf16_round(cyi)
        cbz = _bf16_round(czi)

        def scan_cond(state):
            j, cntv, go = state
            return (j < n_chunks) & go

        def scan_body(state):
            j, cntv, _ = state
            go = jnp.squeeze(lax.slice(cntv, (0,), (1,))) < MAXS
            off = j * LANES
            masks = []
            for u in range(UNROLL):
                sl = pl.ds(off + u * LANES, LANES)
                cp = (cbx * px[sl] + cby * py[sl]) + cbz * pz[sl]
                d2 = (c2i + p2s[sl]) - 2.0 * cp
                masks.append(d2 < RADIUS2)
            for u, m1 in enumerate(masks):
                pos = (cntv + plsc.cumsum(ones, mask=m1)) - 1
                wm = m1 & (pos < MAXS)
                plsc.store_scatter(outv, [rowbase + pos],
                                   iota + (off + u * LANES), mask=wm)
                cntv = cntv + plsc.all_reduce_population_count(m1)
            return j + UNROLL, cntv, go

        _, cntv, _ = lax.while_loop(
            scan_cond, scan_body, (jnp.int32(0), zero, jnp.bool_(True)))

        firstv = plsc.load_gather(outv, [lax.broadcast(rowbase, (LANES,))])
        fillv = jnp.where(cntv > 0, firstv, 0)
        for s0 in range(0, MAXS, LANES):
            slots = iota + s0
            plsc.store_scatter(outv, [rowbase + slots], fillv,
                               mask=slots >= cntv)
        return 0

    lax.fori_loop(0, m_per_w, per_centroid, 0)
    pltpu.sync_copy(outv, out_h.at[wid])


def _ball_query_sc(px, py, pz, cx, cy, cz):
    n = px.shape[1]
    m_per_w = cx.shape[1]
    mesh = plsc.VectorSubcoreMesh(core_axis_name="c", subcore_axis_name="s")
    run = pl.kernel(
        _ball_body,
        out_type=jax.ShapeDtypeStruct((NWORKERS, m_per_w * MAXS), jnp.int32),
        mesh=mesh,
        compiler_params=pltpu.CompilerParams(needs_layout_passes=False),
        scratch_types=[
            pltpu.VMEM((n,), jnp.float32),
            pltpu.VMEM((n,), jnp.float32),
            pltpu.VMEM((n,), jnp.float32),
            pltpu.VMEM((m_per_w,), jnp.float32),
            pltpu.VMEM((m_per_w,), jnp.float32),
            pltpu.VMEM((m_per_w,), jnp.float32),
            pltpu.VMEM((m_per_w * MAXS,), jnp.int32),
            pltpu.VMEM((n,), jnp.float32),
        ],
    )
    return run(px, py, pz, cx, cy, cz)


def kernel(pcs, centroids):
    B, N, _ = pcs.shape
    M = centroids.shape[1]
    m_per_w = (B * M) // NWORKERS
    px = pcs[:, :, 0]
    py = pcs[:, :, 1]
    pz = pcs[:, :, 2]
    cx = centroids[:, :, 0].reshape(NWORKERS, m_per_w)
    cy = centroids[:, :, 1].reshape(NWORKERS, m_per_w)
    cz = centroids[:, :, 2].reshape(NWORKERS, m_per_w)
    out = _ball_query_sc(px, py, pz, cx, cy, cz)
    return out.reshape(B, M, MAXS).astype(jnp.int64)
